# baseline (speedup 1.0000x reference)
import functools

import jax
import jax.numpy as jnp
from jax import lax
from jax.experimental import pallas as pl
from jax.experimental.pallas import tpu as pltpu
from jax.experimental.pallas import tpu_sc as plsc

N_NODES = 10000
N_EDGES = 320000
D = 128

NC = 2
NS = 16
NW = NC * NS
NPAD = 10240
RPT = NPAD // NS
EW = N_EDGES // NW

KP = 80
ITP = EW // KP
BLK = RPT // KP

_sc_mesh = plsc.VectorSubcoreMesh(core_axis_name="c", subcore_axis_name="s")


@functools.partial(
    pl.kernel,
    out_type=(jax.ShapeDtypeStruct((NC * NPAD, D), jnp.float32),
              jax.ShapeDtypeStruct((NC * NPAD, D), jnp.float32)),
    mesh=_sc_mesh,
    scratch_types=[
        pltpu.VMEM((EW,), jnp.int32),
        pltpu.VMEM((ITP, KP), jnp.int32),
        pltpu.VMEM((2, KP, D), jnp.float32),
        pltpu.VMEM((RPT + 16,), jnp.float32),
        pltpu.VMEM((RPT,), jnp.int32),
        pltpu.VMEM((KP,), jnp.float32),
        pltpu.SemaphoreType.DMA,
        pltpu.SemaphoreType.DMA,
        pltpu.SemaphoreType.DMA,
        pltpu.VMEM_SHARED((NPAD,), jnp.float32),
        pltpu.VMEM_SHARED((NPAD, D), jnp.float32),
    ],
)
def _mega_kernel(feat_hbm, src_hbm, dst_hbm, zrow_hbm, rsq_hbm, scl_out,
                 part_out, src_v, dst_v, rows2, dinvt, degi, ones_v, gsem,
                 ssem, isem, deg_sp, acc_sp):
    c = lax.axis_index("c")
    s = lax.axis_index("s")
    wid = s * NC + c
    row0 = s * RPT
    obase = c * NPAD + row0

    pltpu.async_copy(src_hbm.at[wid], src_v, isem)
    for j in range(KP // 16):
        ones_v[pl.ds(j * 16, 16)] = jnp.full((16,), 1.0, jnp.float32)
    pltpu.sync_copy(zrow_hbm.at[pl.ds(row0, RPT)],
                    deg_sp.at[pl.ds(row0, RPT)])
    pltpu.make_async_copy(src_hbm.at[wid], src_v, isem).wait()
    plsc.subcore_barrier()

    def _deg_block(bi):
        pltpu.sync_copy(dst_hbm.at[bi], dst_v)

        def dgroup(g, carry):
            descs = [pltpu.async_copy(ones_v,
                                      deg_sp.at[dst_v.at[g * 5 + k]],
                                      ssem, add=True) for k in range(5)]
            for d in descs:
                d.wait()
            return carry

        lax.fori_loop(0, ITP // 5, dgroup, 0)

    _deg_block(2 * s)
    _deg_block(2 * s + 1)
    pltpu.sync_copy(dst_hbm.at[wid], dst_v)
    plsc.subcore_barrier()

    pltpu.sync_copy(deg_sp.at[pl.ds(row0, RPT)], dinvt.at[pl.ds(0, RPT)])

    def conv_body(k, carry):
        degi[pl.ds(k * 16, 16)] = dinvt[pl.ds(k * 16, 16)].astype(jnp.int32)
        return carry

    lax.fori_loop(0, RPT // 16, conv_body, 0)
    for t in range(BLK):
        pltpu.async_copy(rsq_hbm.at[degi.at[pl.ds(t * KP, KP)]],
                         dinvt.at[pl.ds(t * KP, KP)], gsem).wait()

    for k in range(BLK):
      @pl.when(row0 + k * KP < N_NODES)
      def _blk(k=k):
        pltpu.sync_copy(feat_hbm.at[pl.ds(row0 + k * KP, KP)], rows2.at[0])

        def scale_row(rr, carry, k=k):
            dv = dinvt[pl.ds(k * KP + rr, 16)][0]
            for cc in range(D // 16):
                v = rows2[0, rr, pl.ds(cc * 16, 16)] * dv
                rows2[0, rr, pl.ds(cc * 16, 16)] = v
                rows2[1, rr, pl.ds(cc * 16, 16)] = v * 0.5
            return carry

        lax.fori_loop(0, KP, scale_row, 0)
        pltpu.sync_copy(rows2.at[0], scl_out.at[pl.ds(obase + k * KP, KP)])
        pltpu.sync_copy(rows2.at[1], acc_sp.at[pl.ds(row0 + k * KP, KP)])


    plsc.subcore_barrier()

    def prop_body(j, carry):
        d0 = pltpu.async_copy(
            scl_out.at[src_v.at[pl.ds(2 * j * KP, KP)]], rows2.at[0], gsem)
        d1 = pltpu.async_copy(
            scl_out.at[src_v.at[pl.ds((2 * j + 1) * KP, KP)]], rows2.at[1],
            gsem)
        d0.wait()
        s0 = pltpu.async_copy(rows2.at[0], acc_sp.at[dst_v.at[2 * j]],
                              ssem, add=True)
        d1.wait()
        s1 = pltpu.async_copy(rows2.at[1], acc_sp.at[dst_v.at[2 * j + 1]],
                              ssem, add=True)
        s0.wait()
        s1.wait()
        return carry

    lax.fori_loop(0, ITP // 2, prop_body, 0)
    pltpu.async_copy(scl_out.at[src_v.at[pl.ds((ITP - 1) * KP, KP)]],
                     rows2.at[0], gsem).wait()
    pltpu.sync_copy(rows2.at[0], acc_sp.at[dst_v.at[ITP - 1]], add=True)
    plsc.subcore_barrier()

    for k in range(BLK):
      @pl.when(row0 + k * KP < N_NODES)
      def _pblk(k=k):
        pltpu.sync_copy(acc_sp.at[pl.ds(row0 + k * KP, KP)], rows2.at[0])

        def post_row(rr, carry, k=k):
            dv = dinvt[pl.ds(k * KP + rr, 16)][0]
            for cc in range(D // 16):
                rows2[0, rr, pl.ds(cc * 16, 16)] = (
                    rows2[0, rr, pl.ds(cc * 16, 16)] * dv)
            return carry

        lax.fori_loop(0, KP, post_row, 0)
        pltpu.sync_copy(rows2.at[0], part_out.at[pl.ds(obase + k * KP, KP)])


def _matmul_body(h_ref, x_ref, wh_ref, wx_ref, out_ref):
    w1 = jnp.mean(wh_ref[...], axis=0)
    w2 = jnp.mean(wx_ref[...], axis=0)
    feat = jnp.dot(h_ref[...], w1, preferred_element_type=jnp.float32)
    out_ref[...] = feat + jnp.dot(x_ref[...], w2,
                                  preferred_element_type=jnp.float32)


_BR = 400


def _matmul(h, x, Wh, Wx):
    return pl.pallas_call(
        _matmul_body,
        grid=(N_NODES // _BR,),
        in_specs=[
            pl.BlockSpec((_BR, D), lambda i: (i, 0)),
            pl.BlockSpec((_BR, D), lambda i: (i, 0)),
            pl.BlockSpec((8, D, D), lambda i: (0, 0, 0)),
            pl.BlockSpec((8, D, D), lambda i: (0, 0, 0)),
        ],
        out_specs=pl.BlockSpec((_BR, D), lambda i: (i, 0)),
        out_shape=jax.ShapeDtypeStruct((N_NODES, D), jnp.float32),
    )(h, x, Wh, Wx)


def _final_body(p_ref, bh_ref, bx_ref, out_ref):
    bias = jnp.mean(bh_ref[...] + bx_ref[...], axis=0)
    out_ref[...] = p_ref[0] + p_ref[1] + bias[None, :]


def _final(parts, bh, bx):
    return pl.pallas_call(
        _final_body,
        grid=(N_NODES // _BR,),
        in_specs=[
            pl.BlockSpec((NC, _BR, D), lambda i: (0, i, 0)),
            pl.BlockSpec((8, D), lambda i: (0, 0)),
            pl.BlockSpec((8, D), lambda i: (0, 0)),
        ],
        out_specs=pl.BlockSpec((_BR, D), lambda i: (i, 0)),
        out_shape=jax.ShapeDtypeStruct((N_NODES, D), jnp.float32),
    )(parts, bh, bx)


def kernel(h, x, edge_index, Wh, bh, Wx, bx):
    ei = edge_index.astype(jnp.int32)
    src2 = ei[0].reshape(NW, EW)
    dst3 = ei[1].reshape(NW, ITP, KP)
    zrow = jnp.zeros((NPAD,), jnp.float32)
    rsq = lax.rsqrt(1.0 + jnp.arange(N_EDGES + 8, dtype=jnp.float32))

    feat = _matmul(h, x, Wh, Wx)
    _, parts = _mega_kernel(feat, src2, dst3, zrow, rsq)
    parts = parts.reshape(NC, NPAD, D)
    return _final(parts, bh, bx)


# speedup vs baseline: 1.5162x; 1.5162x over previous
import functools

import jax
import jax.numpy as jnp
from jax import lax
from jax.experimental import pallas as pl
from jax.experimental.pallas import tpu as pltpu
from jax.experimental.pallas import tpu_sc as plsc

N_NODES = 10000
N_EDGES = 320000
D = 128

NC = 2
NS = 16
NW = NC * NS
NPAD = 10240
RPT = NPAD // NS
EW = N_EDGES // NW

KD = 80
ITD = EW // KD
KP = 80
ITP = EW // KP
NBUF = 1

_sc_mesh = plsc.VectorSubcoreMesh(core_axis_name="c", subcore_axis_name="s")


@functools.partial(
    pl.kernel,
    out_type=jax.ShapeDtypeStruct((NC * NPAD,), jnp.float32),
    mesh=_sc_mesh,
    scratch_types=[
        pltpu.VMEM((ITD, KD), jnp.int32),
        pltpu.VMEM((KD,), jnp.float32),
        pltpu.SemaphoreType.DMA,
        pltpu.SemaphoreType.DMA,
        pltpu.VMEM_SHARED((NPAD,), jnp.float32),
    ],
)
def _deg_kernel(dst_hbm, zrow_hbm, deg_out, dst_v, ones_v, isem, ssem,
                deg_sp):
    c = lax.axis_index("c")
    s = lax.axis_index("s")
    wid = s * NC + c
    pltpu.async_copy(dst_hbm.at[wid], dst_v, isem)
    for j in range(KD // 16):
        ones_v[pl.ds(j * 16, 16)] = jnp.full((16,), 1.0, jnp.float32)
    pltpu.sync_copy(zrow_hbm.at[pl.ds(s * RPT, RPT)],
                    deg_sp.at[pl.ds(s * RPT, RPT)])
    pltpu.make_async_copy(dst_hbm.at[wid], dst_v, isem).wait()
    plsc.subcore_barrier()

    def body(g, carry):
        descs = [pltpu.async_copy(ones_v, deg_sp.at[dst_v.at[g * 5 + k]],
                                  ssem, add=True) for k in range(5)]
        for d in descs:
            d.wait()
        return carry

    lax.fori_loop(0, ITD // 5, body, 0)
    plsc.subcore_barrier()
    pltpu.sync_copy(deg_sp.at[pl.ds(s * RPT, RPT)],
                    deg_out.at[pl.ds(c * NPAD + s * RPT, RPT)])


@functools.partial(
    pl.kernel,
    out_type=jax.ShapeDtypeStruct((NC * NPAD, D), jnp.float32),
    mesh=_sc_mesh,
    scratch_types=[
        pltpu.VMEM((EW,), jnp.int32),
        pltpu.VMEM((ITP, KP), jnp.int32),
        pltpu.VMEM((2, KP, D), jnp.float32),
        pltpu.SemaphoreType.DMA,
        pltpu.SemaphoreType.DMA,
        pltpu.SemaphoreType.DMA,
        pltpu.VMEM_SHARED((NPAD, D), jnp.float32),
    ],
)
def _prop_kernel(scaled_hbm, src_hbm, dst_hbm, zacc_hbm, dummy_hbm,
                 acc_out, src_v, dst_v, rows2, gsem, ssem, isem, acc_sp):
    c = lax.axis_index("c")
    s = lax.axis_index("s")
    wid = s * NC + c
    pltpu.async_copy(src_hbm.at[wid], src_v, isem)
    pltpu.async_copy(dst_hbm.at[wid], dst_v, isem)
    pltpu.sync_copy(zacc_hbm.at[pl.ds(s * RPT, RPT)],
                    acc_sp.at[pl.ds(s * RPT, RPT)])
    pltpu.make_async_copy(src_hbm.at[wid], src_v, isem).wait()
    pltpu.make_async_copy(dst_hbm.at[wid], dst_v, isem).wait()
    plsc.subcore_barrier()

    def round_body(j, carry):
        d0 = pltpu.async_copy(
            scaled_hbm.at[src_v.at[pl.ds(2 * j * KP, KP)]],
            rows2.at[0], gsem)
        d1 = pltpu.async_copy(
            scaled_hbm.at[src_v.at[pl.ds((2 * j + 1) * KP, KP)]],
            rows2.at[1], gsem)
        d0.wait()
        s0 = pltpu.async_copy(rows2.at[0], acc_sp.at[dst_v.at[2 * j]],
                              ssem, add=True)
        d1.wait()
        s1 = pltpu.async_copy(rows2.at[1], acc_sp.at[dst_v.at[2 * j + 1]],
                              ssem, add=True)
        s0.wait()
        s1.wait()
        return carry

    lax.fori_loop(0, ITP // 2, round_body, 0)
    pltpu.async_copy(scaled_hbm.at[src_v.at[pl.ds((ITP - 1) * KP, KP)]],
                     rows2.at[0], gsem).wait()
    pltpu.sync_copy(rows2.at[0], acc_sp.at[dst_v.at[ITP - 1]], add=True)

    plsc.subcore_barrier()
    pltpu.sync_copy(acc_sp.at[pl.ds(s * RPT, RPT)],
                    acc_out.at[pl.ds(c * NPAD + s * RPT, RPT)])


def _prep_body(h_ref, x_ref, wh_ref, wx_ref, degt_ref, out_ref):
    w1 = jnp.mean(wh_ref[...], axis=0)
    w2 = jnp.mean(wx_ref[...], axis=0)
    feat = jnp.dot(h_ref[...], w1, preferred_element_type=jnp.float32)
    feat = feat + jnp.dot(x_ref[...], w2, preferred_element_type=jnp.float32)
    deg = degt_ref[:, 0] + degt_ref[:, 1] + 1.0
    dinv = lax.rsqrt(deg)
    out_ref[...] = feat * dinv[:, None]


_BR = 400


def _prep(h, x, Wh, Wx, degt):
    return pl.pallas_call(
        _prep_body,
        grid=(N_NODES // _BR,),
        in_specs=[
            pl.BlockSpec((_BR, D), lambda i: (i, 0)),
            pl.BlockSpec((_BR, D), lambda i: (i, 0)),
            pl.BlockSpec((8, D, D), lambda i: (0, 0, 0)),
            pl.BlockSpec((8, D, D), lambda i: (0, 0, 0)),
            pl.BlockSpec((_BR, NC), lambda i: (i, 0)),
        ],
        out_specs=pl.BlockSpec((_BR, D), lambda i: (i, 0)),
        out_shape=jax.ShapeDtypeStruct((N_NODES, D), jnp.float32),
    )(h, x, Wh, Wx, degt)


def _final_body(acc_ref, scaled_ref, degt_ref, bh_ref, bx_ref, out_ref):
    acc = acc_ref[0] + acc_ref[1] + scaled_ref[...]
    deg = degt_ref[:, 0] + degt_ref[:, 1] + 1.0
    dinv = lax.rsqrt(deg)
    bias = jnp.mean(bh_ref[...] + bx_ref[...], axis=0)
    out_ref[...] = acc * dinv[:, None] + bias[None, :]


def _final(accp, scaled, degt, bh, bx):
    return pl.pallas_call(
        _final_body,
        grid=(N_NODES // _BR,),
        in_specs=[
            pl.BlockSpec((NC, _BR, D), lambda i: (0, i, 0)),
            pl.BlockSpec((_BR, D), lambda i: (i, 0)),
            pl.BlockSpec((_BR, NC), lambda i: (i, 0)),
            pl.BlockSpec((8, D), lambda i: (0, 0)),
            pl.BlockSpec((8, D), lambda i: (0, 0)),
        ],
        out_specs=pl.BlockSpec((_BR, D), lambda i: (i, 0)),
        out_shape=jax.ShapeDtypeStruct((N_NODES, D), jnp.float32),
    )(accp, scaled, degt, bh, bx)


def kernel(h, x, edge_index, Wh, bh, Wx, bx):
    ei = edge_index.astype(jnp.int32)
    src2 = ei[0].reshape(NW, EW)
    dst3p = ei[1].reshape(NW, ITP, KP)
    dst3d = ei[1].reshape(NW, ITD, KD)
    zrow = jnp.zeros((NPAD,), jnp.float32)
    zacc = jnp.zeros((NPAD, D), jnp.float32)
    zdummy = jnp.zeros((KP, D), jnp.float32)

    degp = _deg_kernel(dst3d, zrow)
    degt = degp.reshape(NC, NPAD).T
    scaled = _prep(h, x, Wh, Wx, degt)
    accp = _prop_kernel(scaled, src2, dst3p, zacc, zdummy)
    accp = accp.reshape(NC, NPAD, D)
    return _final(accp, scaled, degt, bh, bx)


# speedup vs baseline: 1.5217x; 1.0036x over previous
import functools

import jax
import jax.numpy as jnp
from jax import lax
from jax.experimental import pallas as pl
from jax.experimental.pallas import tpu as pltpu
from jax.experimental.pallas import tpu_sc as plsc

N_NODES = 10000
N_EDGES = 320000
D = 128

NC = 2
NS = 16
NW = NC * NS
NPAD = 10240
RPT = NPAD // NS
EW = N_EDGES // NW

KD = 80
ITD = EW // KD
KP = 80
ITP = EW // KP
NBUF = 1

_sc_mesh = plsc.VectorSubcoreMesh(core_axis_name="c", subcore_axis_name="s",
                                  num_cores=NC, num_subcores=NS)


@functools.partial(
    pl.kernel,
    out_type=jax.ShapeDtypeStruct((NC * NPAD,), jnp.float32),
    mesh=_sc_mesh,
    scratch_types=[
        pltpu.VMEM((ITD, KD), jnp.int32),
        pltpu.VMEM((KD,), jnp.float32),
        pltpu.SemaphoreType.DMA,
        pltpu.SemaphoreType.DMA,
        pltpu.VMEM_SHARED((NPAD,), jnp.float32),
    ],
)
def _deg_kernel(dst_hbm, zrow_hbm, deg_out, dst_v, ones_v, isem, ssem,
                deg_sp):
    c = lax.axis_index("c")
    s = lax.axis_index("s")
    wid = s * NC + c
    pltpu.async_copy(dst_hbm.at[wid], dst_v, isem)
    for j in range(KD // 16):
        ones_v[pl.ds(j * 16, 16)] = jnp.full((16,), 1.0, jnp.float32)
    pltpu.sync_copy(zrow_hbm.at[pl.ds(s * RPT, RPT)],
                    deg_sp.at[pl.ds(s * RPT, RPT)])
    pltpu.make_async_copy(dst_hbm.at[wid], dst_v, isem).wait()
    plsc.subcore_barrier()

    def body(g, carry):
        descs = [pltpu.async_copy(ones_v, deg_sp.at[dst_v.at[g * 5 + k]],
                                  ssem, add=True) for k in range(5)]
        for d in descs:
            d.wait()
        return carry

    lax.fori_loop(0, ITD // 5, body, 0)
    plsc.subcore_barrier()
    pltpu.sync_copy(deg_sp.at[pl.ds(s * RPT, RPT)],
                    deg_out.at[pl.ds(c * NPAD + s * RPT, RPT)])


@functools.partial(
    pl.kernel,
    out_type=jax.ShapeDtypeStruct((NC * NPAD, D), jnp.float32),
    mesh=_sc_mesh,
    scratch_types=[
        pltpu.VMEM((EW,), jnp.int32),
        pltpu.VMEM((ITP, KP), jnp.int32),
        pltpu.VMEM((2, KP, D), jnp.float32),
        pltpu.SemaphoreType.DMA,
        pltpu.SemaphoreType.DMA,
        pltpu.SemaphoreType.DMA,
        pltpu.VMEM_SHARED((NPAD, D), jnp.float32),
    ],
)
def _prop_kernel(scaled_hbm, src_hbm, dst_hbm, zacc_hbm,
                 acc_out, src_v, dst_v, rows2, gsem, ssem, isem, acc_sp):
    c = lax.axis_index("c")
    s = lax.axis_index("s")
    wid = s * NC + c
    pltpu.async_copy(src_hbm.at[wid], src_v, isem)
    pltpu.async_copy(dst_hbm.at[wid], dst_v, isem)
    pltpu.sync_copy(zacc_hbm.at[pl.ds(s * RPT, RPT)],
                    acc_sp.at[pl.ds(s * RPT, RPT)])
    pltpu.make_async_copy(src_hbm.at[wid], src_v, isem).wait()
    pltpu.make_async_copy(dst_hbm.at[wid], dst_v, isem).wait()
    plsc.subcore_barrier()

    def round_body(j, carry):
        d0 = pltpu.async_copy(
            scaled_hbm.at[src_v.at[pl.ds(2 * j * KP, KP)]],
            rows2.at[0], gsem)
        d1 = pltpu.async_copy(
            scaled_hbm.at[src_v.at[pl.ds((2 * j + 1) * KP, KP)]],
            rows2.at[1], gsem)
        d0.wait()
        s0 = pltpu.async_copy(rows2.at[0], acc_sp.at[dst_v.at[2 * j]],
                              ssem, add=True)
        d1.wait()
        s1 = pltpu.async_copy(rows2.at[1], acc_sp.at[dst_v.at[2 * j + 1]],
                              ssem, add=True)
        s0.wait()
        s1.wait()
        return carry

    lax.fori_loop(0, ITP // 2, round_body, 0)
    pltpu.async_copy(scaled_hbm.at[src_v.at[pl.ds((ITP - 1) * KP, KP)]],
                     rows2.at[0], gsem).wait()
    pltpu.sync_copy(rows2.at[0], acc_sp.at[dst_v.at[ITP - 1]], add=True)

    plsc.subcore_barrier()
    pltpu.sync_copy(acc_sp.at[pl.ds(s * RPT, RPT)],
                    acc_out.at[pl.ds(c * NPAD + s * RPT, RPT)])


def _prep_body(h_ref, x_ref, wh_ref, wx_ref, degt_ref, out_ref):
    w1 = jnp.mean(wh_ref[...], axis=0)
    w2 = jnp.mean(wx_ref[...], axis=0)
    feat = jnp.dot(h_ref[...], w1, preferred_element_type=jnp.float32)
    feat = feat + jnp.dot(x_ref[...], w2, preferred_element_type=jnp.float32)
    deg = degt_ref[:, 0] + degt_ref[:, 1] + 1.0
    dinv = lax.rsqrt(deg)
    out_ref[...] = feat * dinv[:, None]


_BR = 400


def _prep(h, x, Wh, Wx, degt):
    return pl.pallas_call(
        _prep_body,
        grid=(N_NODES // _BR,),
        in_specs=[
            pl.BlockSpec((_BR, D), lambda i: (i, 0)),
            pl.BlockSpec((_BR, D), lambda i: (i, 0)),
            pl.BlockSpec((8, D, D), lambda i: (0, 0, 0)),
            pl.BlockSpec((8, D, D), lambda i: (0, 0, 0)),
            pl.BlockSpec((_BR, NC), lambda i: (i, 0)),
        ],
        out_specs=pl.BlockSpec((_BR, D), lambda i: (i, 0)),
        out_shape=jax.ShapeDtypeStruct((N_NODES, D), jnp.float32),
    )(h, x, Wh, Wx, degt)


def _final_body(acc_ref, scaled_ref, degt_ref, bh_ref, bx_ref, out_ref):
    acc = acc_ref[0] + acc_ref[1] + scaled_ref[...]
    deg = degt_ref[:, 0] + degt_ref[:, 1] + 1.0
    dinv = lax.rsqrt(deg)
    bias = jnp.mean(bh_ref[...] + bx_ref[...], axis=0)
    out_ref[...] = acc * dinv[:, None] + bias[None, :]


def _final(accp, scaled, degt, bh, bx):
    return pl.pallas_call(
        _final_body,
        grid=(N_NODES // _BR,),
        in_specs=[
            pl.BlockSpec((NC, _BR, D), lambda i: (0, i, 0)),
            pl.BlockSpec((_BR, D), lambda i: (i, 0)),
            pl.BlockSpec((_BR, NC), lambda i: (i, 0)),
            pl.BlockSpec((8, D), lambda i: (0, 0)),
            pl.BlockSpec((8, D), lambda i: (0, 0)),
        ],
        out_specs=pl.BlockSpec((_BR, D), lambda i: (i, 0)),
        out_shape=jax.ShapeDtypeStruct((N_NODES, D), jnp.float32),
    )(accp, scaled, degt, bh, bx)


def kernel(h, x, edge_index, Wh, bh, Wx, bx):
    ei = edge_index.astype(jnp.int32)
    src2 = ei[0].reshape(NW, EW)
    dst3p = ei[1].reshape(NW, ITP, KP)
    dst3d = ei[1].reshape(NW, ITD, KD)
    zrow = jnp.zeros((NPAD,), jnp.float32)
    zacc = jnp.zeros((NPAD, D), jnp.float32)

    degp = _deg_kernel(dst3d, zrow)
    degt = degp.reshape(NC, NPAD).T
    scaled = _prep(h, x, Wh, Wx, degt)
    accp = _prop_kernel(scaled, src2, dst3p, zacc)
    accp = accp.reshape(NC, NPAD, D)
    return _final(accp, scaled, degt, bh, bx)
